# Initial kernel scaffold; baseline (speedup 1.0000x reference)
import jax
import jax.numpy as jnp
from jax.experimental import pallas as pl


def kernel(x, edge_index, edge_attr, We1, be1, We2, be2, Wn1, bn1, Wn2, bn2):
    raise NotImplementedError("write your pallas kernel here")


# speedup vs baseline: 2.5005x; 2.5005x over previous
import functools

import jax
import jax.numpy as jnp
from jax import lax
from jax.experimental import pallas as pl
from jax.experimental.pallas import tpu as pltpu
from jax.experimental.pallas import tpu_sc as plsc

N = 10000
E = 320000
D = 128

NC = 2
NS = 16
NW = NC * NS
EPW = E // NW
C = 80
NCHUNK = EPW // C
NP = 10240
NPC = NP // NS
ZR = 128

_sc_mesh = plsc.VectorSubcoreMesh(core_axis_name="c", subcore_axis_name="s")


@functools.partial(
    pl.kernel,
    out_type=jax.ShapeDtypeStruct((E, D), jnp.float32),
    mesh=_sc_mesh,
    scratch_types=[
        pltpu.VMEM((C,), jnp.int32),
        pltpu.VMEM((C,), jnp.int32),
        pltpu.VMEM((C, D), jnp.float32),
        pltpu.VMEM((C, D), jnp.float32),
        pltpu.VMEM((C, D), jnp.float32),
        pltpu.SemaphoreType.DMA,
        pltpu.SemaphoreType.DMA,
    ],
)
def _sc_gather(x_hbm, s_hbm, r_hbm, ea_hbm, out_hbm,
               idx_s, idx_r, ea_v, xs_v, xr_v, sem1, sem2):
    wid = lax.axis_index("s") * NC + lax.axis_index("c")
    base = wid * EPW

    def chunk_body(k, _):
        off = base + k * C
        pltpu.sync_copy(s_hbm.at[pl.ds(off, C)], idx_s)
        pltpu.sync_copy(r_hbm.at[pl.ds(off, C)], idx_r)
        cp_ea = pltpu.async_copy(ea_hbm.at[pl.ds(off, C)], ea_v, sem1)
        cp_s = pltpu.async_copy(x_hbm.at[idx_s], xs_v, sem2)
        cp_r = pltpu.async_copy(x_hbm.at[idx_r], xr_v, sem2)
        cp_ea.wait()
        cp_s.wait()
        cp_r.wait()

        def row_body(i, _):
            for j in range(D // 16):
                sl = pl.ds(j * 16, 16)
                ea_v[i, sl] = ea_v[i, sl] + xs_v[i, sl] + xr_v[i, sl]
            return 0

        lax.fori_loop(0, C, row_body, 0)
        pltpu.sync_copy(ea_v, out_hbm.at[pl.ds(off, C)])
        return 0

    lax.fori_loop(0, NCHUNK, chunk_body, 0)


@functools.partial(
    pl.kernel,
    out_type=jax.ShapeDtypeStruct((NC, NP, D), jnp.float32),
    mesh=_sc_mesh,
    scratch_types=[
        pltpu.VMEM_SHARED((NP, D), jnp.float32),
        pltpu.VMEM((C,), jnp.int32),
        pltpu.VMEM((C, D), jnp.float32),
        pltpu.VMEM((ZR, D), jnp.float32),
        pltpu.SemaphoreType.DMA,
    ],
)
def _sc_scatter(ne_hbm, r_hbm, out_hbm, agg_sh, idx_v, rows_v, zbuf, sem):
    cid = lax.axis_index("c")
    sid = lax.axis_index("s")
    wid = sid * NC + cid
    base = wid * EPW

    zeros = jnp.zeros((16,), jnp.float32)

    def zrow(i, _):
        for j in range(D // 16):
            zbuf[i, pl.ds(j * 16, 16)] = zeros
        return 0

    lax.fori_loop(0, ZR, zrow, 0)
    for t in range(NPC // ZR):
        pltpu.sync_copy(zbuf, agg_sh.at[pl.ds(sid * NPC + t * ZR, ZR)])
    plsc.subcore_barrier()

    def chunk_body(k, _):
        off = base + k * C
        pltpu.sync_copy(r_hbm.at[pl.ds(off, C)], idx_v)
        pltpu.sync_copy(ne_hbm.at[pl.ds(off, C)], rows_v)
        pltpu.sync_copy(rows_v, agg_sh.at[idx_v], add=True)
        return 0

    lax.fori_loop(0, NCHUNK, chunk_body, 0)
    plsc.subcore_barrier()

    pltpu.sync_copy(agg_sh.at[pl.ds(sid * NPC, NPC)],
                    out_hbm.at[cid].at[pl.ds(sid * NPC, NPC)])


def _edge_mlp_body(e_ref, w1_ref, b1_ref, w2_ref, b2_ref, o_ref):
    h = jnp.dot(e_ref[...], w1_ref[...], preferred_element_type=jnp.float32)
    h = jnp.maximum(h + b1_ref[...], 0.0)
    o_ref[...] = (
        jnp.dot(h, w2_ref[...], preferred_element_type=jnp.float32)
        + b2_ref[...]
    )


def _node_mlp_body(x_ref, p0_ref, p1_ref, w1_ref, b1_ref, w2_ref, b2_ref,
                   o_ref):
    n = x_ref[...] + p0_ref[0] + p1_ref[0]
    h = jnp.dot(n, w1_ref[...], preferred_element_type=jnp.float32)
    h = jnp.maximum(h + b1_ref[...], 0.0)
    o_ref[...] = (
        jnp.dot(h, w2_ref[...], preferred_element_type=jnp.float32)
        + b2_ref[...]
    )


_BE = 1280
_BN = 1000


def _full(shape):
    return pl.BlockSpec(shape, lambda i: (0,) * len(shape))


def _edge_mlp(e_in, We1, be1, We2, be2):
    return pl.pallas_call(
        _edge_mlp_body,
        grid=(E // _BE,),
        in_specs=[
            pl.BlockSpec((_BE, D), lambda i: (i, 0)),
            _full((D, D)), _full((1, D)), _full((D, D)), _full((1, D)),
        ],
        out_specs=pl.BlockSpec((_BE, D), lambda i: (i, 0)),
        out_shape=jax.ShapeDtypeStruct((E, D), jnp.float32),
    )(e_in, We1, be1.reshape(1, D), We2, be2.reshape(1, D))


def _node_mlp(x, parts, Wn1, bn1, Wn2, bn2):
    return pl.pallas_call(
        _node_mlp_body,
        grid=(N // _BN,),
        in_specs=[
            pl.BlockSpec((_BN, D), lambda i: (i, 0)),
            pl.BlockSpec((1, _BN, D), lambda i: (0, i, 0)),
            pl.BlockSpec((1, _BN, D), lambda i: (1, i, 0)),
            _full((D, D)), _full((1, D)), _full((D, D)), _full((1, D)),
        ],
        out_specs=pl.BlockSpec((_BN, D), lambda i: (i, 0)),
        out_shape=jax.ShapeDtypeStruct((N, D), jnp.float32),
    )(x, parts, parts, Wn1, bn1.reshape(1, D), Wn2, bn2.reshape(1, D))


def kernel(x, edge_index, edge_attr, We1, be1, We2, be2, Wn1, bn1, Wn2, bn2):
    senders = edge_index[0]
    receivers = edge_index[1]
    e_in = _sc_gather(x, senders, receivers, edge_attr)
    new_edge = _edge_mlp(e_in, We1, be1, We2, be2)
    parts = _sc_scatter(new_edge, receivers)[:, :N]
    new_node = _node_mlp(x, parts, Wn1, bn1, Wn2, bn2)
    return new_node, new_edge


# speedup vs baseline: 4.2694x; 1.7074x over previous
import functools

import jax
import jax.numpy as jnp
from jax import lax
from jax.experimental import pallas as pl
from jax.experimental.pallas import tpu as pltpu
from jax.experimental.pallas import tpu_sc as plsc

N = 10000
E = 320000
D = 128

NC = 2
NS = 16
NW = NC * NS
EPW = E // NW
C = 80
NCHUNK = EPW // C
NP = 10240
NPC = NP // NS
ZR = 128

_sc_mesh = plsc.VectorSubcoreMesh(core_axis_name="c", subcore_axis_name="s")


@functools.partial(
    pl.kernel,
    out_type=jax.ShapeDtypeStruct((E, D), jnp.float32),
    mesh=_sc_mesh,
    scratch_types=[
        pltpu.VMEM((EPW,), jnp.int32),
        pltpu.VMEM((EPW,), jnp.int32),
        [pltpu.VMEM((C, D), jnp.float32)] * 2,
        [pltpu.VMEM((C, D), jnp.float32)] * 2,
        [pltpu.VMEM((C, D), jnp.float32)] * 2,
        [pltpu.VMEM((C, D), jnp.float32)] * 2,
        [pltpu.SemaphoreType.DMA] * 2,
        [pltpu.SemaphoreType.DMA] * 2,
        pltpu.SemaphoreType.DMA,
    ],
)
def _sc_gather(x_hbm, s_hbm, r_hbm, ea_hbm, out_hbm,
               idx_s, idx_r, ea_v, xs_v, xr_v, o_v, sem_in, sem_out,
               sem_idx):
    wid = lax.axis_index("s") * NC + lax.axis_index("c")
    base = wid * EPW

    cp_s = pltpu.async_copy(s_hbm.at[pl.ds(base, EPW)], idx_s, sem_idx)
    cp_r = pltpu.async_copy(r_hbm.at[pl.ds(base, EPW)], idx_r, sem_idx)
    cp_s.wait()
    cp_r.wait()

    def issue_in(s, k):
        off = base + k * C
        ioff = k * C
        pltpu.async_copy(ea_hbm.at[pl.ds(off, C)], ea_v[s], sem_in[s])
        pltpu.async_copy(x_hbm.at[idx_s.at[pl.ds(ioff, C)]], xs_v[s],
                         sem_in[s])
        pltpu.async_copy(x_hbm.at[idx_r.at[pl.ds(ioff, C)]], xr_v[s],
                         sem_in[s])

    def wait_in(s):
        pltpu.make_async_copy(ea_hbm.at[pl.ds(0, C)], ea_v[s],
                              sem_in[s]).wait()
        pltpu.make_async_copy(ea_hbm.at[pl.ds(0, C)], xs_v[s],
                              sem_in[s]).wait()
        pltpu.make_async_copy(ea_hbm.at[pl.ds(0, C)], xr_v[s],
                              sem_in[s]).wait()

    def wait_out(s):
        pltpu.make_async_copy(o_v[s], out_hbm.at[pl.ds(0, C)],
                              sem_out[s]).wait()

    def add_and_store(s, k):
        def row_body(i, _):
            for j in range(D // 16):
                sl = pl.ds(j * 16, 16)
                o_v[s][i, sl] = ea_v[s][i, sl] + xs_v[s][i, sl] + xr_v[s][i, sl]
            return 0

        lax.fori_loop(0, C, row_body, 0)
        pltpu.async_copy(o_v[s], out_hbm.at[pl.ds(base + k * C, C)],
                         sem_out[s])

    issue_in(0, 0)
    issue_in(1, 1)

    def pair_body(j, _):
        k0 = 2 * j
        wait_in(0)

        @pl.when(j >= 1)
        def _():
            wait_out(0)

        add_and_store(0, k0)
        issue_in(0, k0 + 2)

        wait_in(1)

        @pl.when(j >= 1)
        def _():
            wait_out(1)

        add_and_store(1, k0 + 1)

        @pl.when(j < (NCHUNK - 1) // 2 - 1)
        def _():
            issue_in(1, k0 + 3)

        return 0

    lax.fori_loop(0, (NCHUNK - 1) // 2, pair_body, 0)
    wait_in(0)
    wait_out(0)
    add_and_store(0, NCHUNK - 1)
    wait_out(1)
    wait_out(0)


@functools.partial(
    pl.kernel,
    out_type=jax.ShapeDtypeStruct((NC, NP, D), jnp.float32),
    mesh=_sc_mesh,
    scratch_types=[
        pltpu.VMEM_SHARED((NP, D), jnp.float32),
        [pltpu.VMEM((C,), jnp.int32)] * 2,
        [pltpu.VMEM((C, D), jnp.float32)] * 2,
        pltpu.VMEM((ZR, D), jnp.float32),
        [pltpu.SemaphoreType.DMA] * 2,
    ],
)
def _sc_scatter(ne_hbm, r_hbm, out_hbm, agg_sh, idx_v, rows_v, zbuf,
                sem_ld):
    cid = lax.axis_index("c")
    sid = lax.axis_index("s")
    wid = sid * NC + cid
    base = wid * EPW

    def issue_ld(s, k):
        off = base + k * C
        pltpu.async_copy(r_hbm.at[pl.ds(off, C)], idx_v[s], sem_ld[s])
        pltpu.async_copy(ne_hbm.at[pl.ds(off, C)], rows_v[s], sem_ld[s])

    def wait_ld(s):
        pltpu.make_async_copy(r_hbm.at[pl.ds(0, C)], idx_v[s],
                              sem_ld[s]).wait()
        pltpu.make_async_copy(ne_hbm.at[pl.ds(0, C)], rows_v[s],
                              sem_ld[s]).wait()

    def scat(s, k):
        pltpu.sync_copy(rows_v[s], agg_sh.at[idx_v[s]], add=True)

    issue_ld(0, 0)
    issue_ld(1, 1)

    zeros = jnp.zeros((16,), jnp.float32)

    def zrow(i, _):
        for j in range(D // 16):
            zbuf[i, pl.ds(j * 16, 16)] = zeros
        return 0

    lax.fori_loop(0, ZR, zrow, 0)
    for t in range(NPC // ZR):
        pltpu.sync_copy(zbuf, agg_sh.at[pl.ds(sid * NPC + t * ZR, ZR)])
    plsc.subcore_barrier()

    def pair_body(j, _):
        k0 = 2 * j
        wait_ld(0)
        scat(0, k0)
        issue_ld(0, k0 + 2)
        wait_ld(1)
        scat(1, k0 + 1)

        @pl.when(j < (NCHUNK - 1) // 2 - 1)
        def _():
            issue_ld(1, k0 + 3)

        return 0

    lax.fori_loop(0, (NCHUNK - 1) // 2, pair_body, 0)
    wait_ld(0)
    scat(0, NCHUNK - 1)
    plsc.subcore_barrier()

    pltpu.sync_copy(agg_sh.at[pl.ds(sid * NPC, NPC)],
                    out_hbm.at[cid].at[pl.ds(sid * NPC, NPC)])


def _edge_mlp_body(e_ref, w1_ref, b1_ref, w2_ref, b2_ref, o_ref):
    h = jnp.dot(e_ref[...], w1_ref[...], preferred_element_type=jnp.float32)
    h = jnp.maximum(h + b1_ref[...], 0.0)
    o_ref[...] = (
        jnp.dot(h, w2_ref[...], preferred_element_type=jnp.float32)
        + b2_ref[...]
    )


def _node_mlp_body(x_ref, p0_ref, p1_ref, w1_ref, b1_ref, w2_ref, b2_ref,
                   o_ref):
    n = x_ref[...] + p0_ref[0] + p1_ref[0]
    h = jnp.dot(n, w1_ref[...], preferred_element_type=jnp.float32)
    h = jnp.maximum(h + b1_ref[...], 0.0)
    o_ref[...] = (
        jnp.dot(h, w2_ref[...], preferred_element_type=jnp.float32)
        + b2_ref[...]
    )


_BE = 1280
_BN = 1000


def _full(shape):
    return pl.BlockSpec(shape, lambda i: (0,) * len(shape))


def _edge_mlp(e_in, We1, be1, We2, be2):
    return pl.pallas_call(
        _edge_mlp_body,
        grid=(E // _BE,),
        in_specs=[
            pl.BlockSpec((_BE, D), lambda i: (i, 0)),
            _full((D, D)), _full((1, D)), _full((D, D)), _full((1, D)),
        ],
        out_specs=pl.BlockSpec((_BE, D), lambda i: (i, 0)),
        out_shape=jax.ShapeDtypeStruct((E, D), jnp.float32),
    )(e_in, We1, be1.reshape(1, D), We2, be2.reshape(1, D))


def _node_mlp(x, parts, Wn1, bn1, Wn2, bn2):
    return pl.pallas_call(
        _node_mlp_body,
        grid=(N // _BN,),
        in_specs=[
            pl.BlockSpec((_BN, D), lambda i: (i, 0)),
            pl.BlockSpec((1, _BN, D), lambda i: (0, i, 0)),
            pl.BlockSpec((1, _BN, D), lambda i: (1, i, 0)),
            _full((D, D)), _full((1, D)), _full((D, D)), _full((1, D)),
        ],
        out_specs=pl.BlockSpec((_BN, D), lambda i: (i, 0)),
        out_shape=jax.ShapeDtypeStruct((N, D), jnp.float32),
    )(x, parts, parts, Wn1, bn1.reshape(1, D), Wn2, bn2.reshape(1, D))


def kernel(x, edge_index, edge_attr, We1, be1, We2, be2, Wn1, bn1, Wn2, bn2):
    senders = edge_index[0]
    receivers = edge_index[1]
    e_in = _sc_gather(x, senders, receivers, edge_attr)
    new_edge = _edge_mlp(e_in, We1, be1, We2, be2)
    parts = _sc_scatter(new_edge, receivers)[:, :N]
    new_node = _node_mlp(x, parts, Wn1, bn1, Wn2, bn2)
    return new_node, new_edge
